# baseline (speedup 1.0000x reference)
import jax
import jax.numpy as jnp
from jax.experimental import pallas as pl
from jax.experimental.pallas import tpu as pltpu
from jax.experimental.pallas import tpu_sc as plsc

_LANES = 16


def kernel(x, W_veh, W_time, W_pos):
    B, N, T, F = x.shape
    D = W_veh.shape[1]
    mesh = plsc.VectorSubcoreMesh(core_axis_name="c", subcore_axis_name="s")

    @pl.kernel(
        out_type=jax.ShapeDtypeStruct((N, T, D, B), W_veh.dtype),
        mesh=mesh,
    )
    def sc_kernel(wv_hbm, wt_hbm, o_hbm):
        def body(wv_b, wt_b, o_b):
            @pl.loop(0, D, step=_LANES)
            def _(dc):
                v = wv_b[0, pl.ds(dc, _LANES)] + wt_b[0, pl.ds(dc, _LANES)]
                for j in range(_LANES):
                    row = jnp.full((_LANES,), v[j], jnp.float32)

                    @pl.loop(0, B, step=_LANES)
                    def _(bc):
                        o_b[0, 0, dc + j, pl.ds(bc, _LANES)] = row

        pltpu.emit_pipeline(
            body,
            grid=(N, T),
            in_specs=[
                pl.BlockSpec((1, D), lambda n, t: (n, 0)),
                pl.BlockSpec((1, D), lambda n, t: (t, 0)),
            ],
            out_specs=[pl.BlockSpec((1, 1, D, B), lambda n, t: (n, t, 0, 0))],
            core_axis_name=("c", "s"),
            dimension_semantics=(pltpu.PARALLEL, pltpu.PARALLEL),
        )(wv_hbm, wt_hbm, o_hbm)

    out = sc_kernel(W_veh[:N], W_time[:T])
    return jnp.transpose(out, (3, 0, 1, 2))


# speedup vs baseline: 2.1313x; 2.1313x over previous
import jax
import jax.numpy as jnp
from jax.experimental import pallas as pl
from jax.experimental.pallas import tpu as pltpu

_BN = 2


def _st_embed_kernel(wv_ref, wt_ref, out_ref, bt_ref):
    T, D = wt_ref.shape
    B = out_ref.shape[3]
    i = pl.program_id(0)

    @pl.when(i == 0)
    def _init():
        bt_ref[...] = jnp.broadcast_to(wt_ref[...][:, :, None], bt_ref.shape)

    bt = bt_ref[...]
    for j in range(_BN):
        a_n = jnp.broadcast_to(wv_ref[pl.ds(i * _BN + j, 1), :][0][:, None], (D, B))
        out_ref[j] = bt + jnp.broadcast_to(a_n[None], (T, D, B))


def kernel(x, W_veh, W_time, W_pos):
    B, N, T, F = x.shape
    D = W_veh.shape[1]
    out = pl.pallas_call(
        _st_embed_kernel,
        grid=(N // _BN,),
        in_specs=[
            pl.BlockSpec((N, D), lambda i: (0, 0)),
            pl.BlockSpec((T, D), lambda i: (0, 0)),
        ],
        out_specs=pl.BlockSpec((_BN, T, D, B), lambda i: (i, 0, 0, 0)),
        out_shape=jax.ShapeDtypeStruct((N, T, D, B), W_veh.dtype),
        scratch_shapes=[
            pltpu.VMEM((T, D, B), W_veh.dtype),
        ],
    )(W_veh[:N], W_time[:T])
    return jnp.transpose(out, (3, 0, 1, 2))
